# Initial kernel scaffold; baseline (speedup 1.0000x reference)
import jax
import jax.numpy as jnp
from jax.experimental import pallas as pl


def kernel(feature_map):
    raise NotImplementedError("write your pallas kernel here")


# speedup vs baseline: 1.3835x; 1.3835x over previous
import functools

import jax
import jax.numpy as jnp
from jax import lax
from jax.experimental import pallas as pl
from jax.experimental.pallas import tpu as pltpu
from jax.experimental.pallas import tpu_sc as plsc

C, D, H, W = 64, 32, 128, 128
PD, PH, PW = 8, 16, 16
ND, NH, NW_ = D // PD, H // PH, W // PW
NSEL = 128
NUM_CORES, NUM_SUBCORES = 2, 16
NWORK = NUM_CORES * NUM_SUBCORES
PATCHES_PER_WORKER = NSEL // NWORK
CC = 16
NCHUNK = C // CC


def _body(fm, out_p, out_c, buf0, buf1, cbuf, sin0, sin1, sout0, sout1):
    wid = lax.axis_index("c") * NUM_SUBCORES + lax.axis_index("s")
    bufs = (buf0, buf1)
    sins = (sin0, sin1)
    souts = (sout0, sout1)
    pending = [None, None]

    k = 0
    for j in range(PATCHES_PER_WORKER):
        n = wid * PATCHES_PER_WORKER + j
        sel = (n * 255) // 127
        pdi = sel // (NH * NW_)
        phi = (sel // NW_) % NH
        pwi = sel % NW_
        d0 = pdi * PD
        h0 = phi * PH
        w0 = pwi * PW
        for cc in range(NCHUNK):
            b = k % 2
            if pending[b] is not None:
                pending[b].wait()
            c0 = cc * CC
            src = fm.at[pl.ds(c0, CC), pl.ds(d0, PD), pl.ds(h0, PH), pl.ds(w0, PW)]
            gather = pltpu.make_async_copy(src, bufs[b], sins[b])
            gather.start()
            gather.wait()
            put = pltpu.make_async_copy(bufs[b], out_p.at[n, pl.ds(c0, CC)], souts[b])
            put.start()
            pending[b] = put
            k += 1
    for b in range(2):
        if pending[b] is not None:
            pending[b].wait()

    @pl.when(wid == 0)
    def _():
        for v in range(NSEL // 16):
            nvec = lax.iota(jnp.int32, 16) + v * 16
            sv = (nvec << 1) + ((nvec + 1) >> 7)
            cbuf[pl.ds(v * 16, 16)] = (sv >> 6) << 3
            cbuf[pl.ds(NSEL + v * 16, 16)] = ((sv >> 3) & 7) << 4
            cbuf[pl.ds(2 * NSEL + v * 16, 16)] = (sv & 7) << 4
        pltpu.sync_copy(cbuf, out_c)


@jax.jit
def kernel(feature_map):
    mesh = plsc.VectorSubcoreMesh(
        core_axis_name="c", subcore_axis_name="s",
        num_cores=NUM_CORES, num_subcores=NUM_SUBCORES)
    patches, coords_flat = pl.kernel(
        _body,
        out_type=(
            jax.ShapeDtypeStruct((NSEL, C, PD, PH, PW), jnp.float32),
            jax.ShapeDtypeStruct((NSEL * 3,), jnp.int32),
        ),
        mesh=mesh,
        compiler_params=pltpu.CompilerParams(use_tc_tiling_on_sc=False),
        scratch_types=(
            pltpu.VMEM((CC, PD, PH, PW), jnp.float32),
            pltpu.VMEM((CC, PD, PH, PW), jnp.float32),
            pltpu.VMEM((NSEL * 3,), jnp.int32),
            pltpu.SemaphoreType.DMA,
            pltpu.SemaphoreType.DMA,
            pltpu.SemaphoreType.DMA,
            pltpu.SemaphoreType.DMA,
        ),
    )(feature_map)
    return patches, coords_flat.reshape(3, NSEL).T
